# Initial kernel scaffold; baseline (speedup 1.0000x reference)
import jax
import jax.numpy as jnp
from jax.experimental import pallas as pl


def kernel(h, edge_index, aW1, ab1, aW2, ab2, sW1, sb1, sW2, sb2):
    raise NotImplementedError("write your pallas kernel here")


# speedup vs baseline: 20.0693x; 20.0693x over previous
import functools

import jax
import jax.numpy as jnp
from jax import lax
from jax.experimental import pallas as pl
from jax.experimental.pallas import tpu as pltpu
from jax.experimental.pallas import tpu_sc as plsc

N = 10000
E = 320000
D = 128
NC, NS = 2, 16
NW = NC * NS
C = 100
CHUNKS = E // (NW * C)
RPS = 632
RPS_LAST = N - (NS - 1) * RPS
DEGW = 16

_MESH = plsc.VectorSubcoreMesh(core_axis_name="c", subcore_axis_name="s")


def _sc_degree(dstr, ones, zeros16):
    @functools.partial(
        pl.kernel,
        mesh=_MESH,
        out_type=jax.ShapeDtypeStruct((NC, N, DEGW), jnp.float32),
        scratch_types=[
            pltpu.VMEM((CHUNKS, C), jnp.int32),
            pltpu.VMEM((C, DEGW), jnp.float32),
            pltpu.VMEM_SHARED((N, DEGW), jnp.float32),
        ],
    )
    def k(dstr_hbm, ones_hbm, zeros_hbm, out_hbm, dst_idx, ones_v, acc):
        cid = lax.axis_index("c")
        sid = lax.axis_index("s")
        wid = sid * NC + cid
        pltpu.sync_copy(dstr_hbm.at[wid], dst_idx)
        pltpu.sync_copy(ones_hbm, ones_v)
        r0 = sid * RPS

        @pl.when(sid < NS - 1)
        def _():
            pltpu.sync_copy(zeros_hbm.at[pl.ds(r0, RPS)],
                            acc.at[pl.ds(r0, RPS)])

        @pl.when(sid == NS - 1)
        def _():
            pltpu.sync_copy(zeros_hbm.at[pl.ds(r0, RPS_LAST)],
                            acc.at[pl.ds(r0, RPS_LAST)])

        plsc.subcore_barrier()

        def body(j, carry):
            pltpu.sync_copy(ones_v, acc.at[dst_idx.at[j]], add=True)
            return carry

        lax.fori_loop(0, CHUNKS, body, 0)
        plsc.subcore_barrier()

        @pl.when(sid < NS - 1)
        def _():
            pltpu.sync_copy(acc.at[pl.ds(r0, RPS)],
                            out_hbm.at[cid, pl.ds(r0, RPS)])

        @pl.when(sid == NS - 1)
        def _():
            pltpu.sync_copy(acc.at[pl.ds(r0, RPS_LAST)],
                            out_hbm.at[cid, pl.ds(r0, RPS_LAST)])

    return k(dstr, ones, zeros16)


def _sc_edge_apply(y, srcr, dstr, zeros):
    @functools.partial(
        pl.kernel,
        mesh=_MESH,
        out_type=jax.ShapeDtypeStruct((NC, N, D), jnp.float32),
        scratch_types=[
            pltpu.VMEM((CHUNKS, C), jnp.int32),
            pltpu.VMEM((CHUNKS, C), jnp.int32),
            pltpu.VMEM((C, D), jnp.float32),
            pltpu.VMEM_SHARED((N, D), jnp.float32),
        ],
    )
    def k(y_hbm, srcr_hbm, dstr_hbm, zeros_hbm, out_hbm,
          src_idx, dst_idx, rows_v, acc):
        cid = lax.axis_index("c")
        sid = lax.axis_index("s")
        wid = sid * NC + cid
        pltpu.sync_copy(srcr_hbm.at[wid], src_idx)
        pltpu.sync_copy(dstr_hbm.at[wid], dst_idx)
        r0 = sid * RPS

        @pl.when(sid < NS - 1)
        def _():
            pltpu.sync_copy(zeros_hbm.at[pl.ds(r0, RPS)],
                            acc.at[pl.ds(r0, RPS)])

        @pl.when(sid == NS - 1)
        def _():
            pltpu.sync_copy(zeros_hbm.at[pl.ds(r0, RPS_LAST)],
                            acc.at[pl.ds(r0, RPS_LAST)])

        plsc.subcore_barrier()

        def body(j, carry):
            pltpu.sync_copy(y_hbm.at[src_idx.at[j]], rows_v)
            pltpu.sync_copy(rows_v, acc.at[dst_idx.at[j]], add=True)
            return carry

        lax.fori_loop(0, CHUNKS, body, 0)
        plsc.subcore_barrier()

        @pl.when(sid < NS - 1)
        def _():
            pltpu.sync_copy(acc.at[pl.ds(r0, RPS)],
                            out_hbm.at[cid, pl.ds(r0, RPS)])

        @pl.when(sid == NS - 1)
        def _():
            pltpu.sync_copy(acc.at[pl.ds(r0, RPS_LAST)],
                            out_hbm.at[cid, pl.ds(r0, RPS_LAST)])

    return k(y, srcr, dstr, zeros)


_TM = 2000


def _tc_prepare(degp, h):
    def body(degp_ref, h_ref, y1_ref, dinv_ref):
        deg = degp_ref[0] + degp_ref[1] + 1.0
        dinv = lax.rsqrt(deg)
        dinv_ref[...] = dinv
        y1_ref[...] = dinv[:, 0:1] * h_ref[...]

    return pl.pallas_call(
        body,
        grid=(N // _TM,),
        in_specs=[
            pl.BlockSpec((NC, _TM, DEGW), lambda i: (0, i, 0)),
            pl.BlockSpec((_TM, D), lambda i: (i, 0)),
        ],
        out_specs=[
            pl.BlockSpec((_TM, D), lambda i: (i, 0)),
            pl.BlockSpec((_TM, DEGW), lambda i: (i, 0)),
        ],
        out_shape=[
            jax.ShapeDtypeStruct((N, D), jnp.float32),
            jax.ShapeDtypeStruct((N, DEGW), jnp.float32),
        ],
    )(degp, h)


def _tc_mid(zp, y1, dinv16, W1c, b1c):
    def body(zp_ref, y1_ref, dinv_ref, w_ref, b_ref, y2_ref):
        dinv = dinv_ref[:, 0:1]
        zz = dinv * (zp_ref[0] + zp_ref[1] + y1_ref[...])
        t = jnp.maximum(
            lax.dot_general(zz, w_ref[...], (((1,), (0,)), ((), ())),
                            preferred_element_type=jnp.float32)
            + b_ref[...], 0.0)
        y2_ref[...] = dinv * t

    return pl.pallas_call(
        body,
        grid=(N // _TM,),
        in_specs=[
            pl.BlockSpec((NC, _TM, D), lambda i: (0, i, 0)),
            pl.BlockSpec((_TM, D), lambda i: (i, 0)),
            pl.BlockSpec((_TM, DEGW), lambda i: (i, 0)),
            pl.BlockSpec((D, D), lambda i: (0, 0)),
            pl.BlockSpec((1, D), lambda i: (0, 0)),
        ],
        out_specs=pl.BlockSpec((_TM, D), lambda i: (i, 0)),
        out_shape=jax.ShapeDtypeStruct((N, D), jnp.float32),
    )(zp, y1, dinv16, W1c, b1c)


def _tc_out(zp, y2, dinv16, aW2, ab2, sW2, sb2):
    def body(zp_ref, y2_ref, dinv_ref, aw_ref, ab_ref, sw_ref, sb_ref,
             x_ref, h_ref):
        dinv = dinv_ref[:, 0:1]
        au = dinv * (zp_ref[0] + zp_ref[1] + y2_ref[...])
        x_ref[...] = lax.dot_general(
            au[:, :64], aw_ref[...], (((1,), (0,)), ((), ())),
            preferred_element_type=jnp.float32) + ab_ref[...]
        h_ref[...] = lax.dot_general(
            au[:, 64:], sw_ref[...], (((1,), (0,)), ((), ())),
            preferred_element_type=jnp.float32) + sb_ref[...]

    return pl.pallas_call(
        body,
        grid=(N // _TM,),
        in_specs=[
            pl.BlockSpec((NC, _TM, D), lambda i: (0, i, 0)),
            pl.BlockSpec((_TM, D), lambda i: (i, 0)),
            pl.BlockSpec((_TM, DEGW), lambda i: (i, 0)),
            pl.BlockSpec((64, D), lambda i: (0, 0)),
            pl.BlockSpec((1, D), lambda i: (0, 0)),
            pl.BlockSpec((64, D), lambda i: (0, 0)),
            pl.BlockSpec((1, D), lambda i: (0, 0)),
        ],
        out_specs=[
            pl.BlockSpec((_TM, D), lambda i: (i, 0)),
            pl.BlockSpec((_TM, D), lambda i: (i, 0)),
        ],
        out_shape=[
            jax.ShapeDtypeStruct((N, D), jnp.float32),
            jax.ShapeDtypeStruct((N, D), jnp.float32),
        ],
    )(zp, y2, dinv16, aW2, ab2, sW2, sb2)


_SM = 1024


def _tc_gram(h_):
    def body(a_ref, b_ref, o_ref):
        o_ref[...] = lax.dot_general(
            a_ref[...], b_ref[...], (((1,), (1,)), ((), ())),
            preferred_element_type=jnp.float32)

    g = pl.cdiv(N, _SM)
    return pl.pallas_call(
        body,
        grid=(g, g),
        in_specs=[
            pl.BlockSpec((_SM, D), lambda i, j: (i, 0)),
            pl.BlockSpec((_SM, D), lambda i, j: (j, 0)),
        ],
        out_specs=pl.BlockSpec((_SM, _SM), lambda i, j: (i, j)),
        out_shape=jax.ShapeDtypeStruct((N, N), jnp.float32),
    )(h_, h_)


def kernel(h, edge_index, aW1, ab1, aW2, ab2, sW1, sb1, sW2, sb2):
    srcr = edge_index[0].reshape(NW, CHUNKS, C)
    dstr = edge_index[1].reshape(NW, CHUNKS, C)
    zeros = jnp.zeros((N, D), jnp.float32)
    zeros16 = jnp.zeros((N, DEGW), jnp.float32)
    ones = jnp.ones((C, DEGW), jnp.float32)
    W1c = jnp.concatenate([aW1, sW1], axis=1)
    b1c = jnp.concatenate([ab1, sb1]).reshape(1, D)

    degp = _sc_degree(dstr, ones, zeros16)
    y1, dinv16 = _tc_prepare(degp, h)
    z1p = _sc_edge_apply(y1, srcr, dstr, zeros)
    y2 = _tc_mid(z1p, y1, dinv16, W1c, b1c)
    z2p = _sc_edge_apply(y2, srcr, dstr, zeros)
    x_, h_ = _tc_out(z2p, y2, dinv16, aW2, ab2.reshape(1, D),
                     sW2, sb2.reshape(1, D))
    s_ = _tc_gram(h_)
    return (x_, s_, h)
